# baseline (speedup 1.0000x reference)
import jax
import jax.numpy as jnp
from jax import lax
from jax.experimental import pallas as pl
from jax.experimental.pallas import tpu as pltpu


def _gcn_body(w1t_ref, b1_ref, w2_ref, b2c_ref, x_hbm, adj_hbm, h_ref,
              abuf, abf, xbuf, g2buf, asems, xsems):
    B = h_ref.shape[0]

    def start_batch(i):
        pltpu.make_async_copy(
            adj_hbm.at[i], abuf.at[i % 4], asems.at[i % 4]).start()
        pltpu.make_async_copy(
            x_hbm.at[i], xbuf.at[i % 3], xsems.at[i % 3]).start()

    def wait_batch(i):
        pltpu.make_async_copy(
            adj_hbm.at[i], abuf.at[i % 4], asems.at[i % 4]).wait()
        pltpu.make_async_copy(
            x_hbm.at[i], xbuf.at[i % 3], xsems.at[i % 3]).wait()

    start_batch(0)
    if B > 1:
        start_batch(1)
    for s in range(B + 1):
        if s < B:
            if s + 2 < B:
                start_batch(s + 2)
            wait_batch(s)
            ab = abuf[s % 4].astype(jnp.bfloat16)
            abf[s % 2] = ab
            g1 = jnp.dot(xbuf[s % 3], w1t_ref[...],
                         preferred_element_type=jnp.float32) + b1_ref[...]
            h1t = lax.dot_general(g1.astype(jnp.bfloat16), ab,
                                  (((0,), (1,)), ((), ())),
                                  preferred_element_type=jnp.float32)
            r1t = jnp.maximum(h1t, 0.0)
            g2buf[s % 2] = jnp.dot(
                w2_ref[...], r1t,
                preferred_element_type=jnp.float32) + b2c_ref[...]
        if s >= 1:
            h2t = lax.dot_general(
                g2buf[(s - 1) % 2].astype(jnp.bfloat16), abf[(s - 1) % 2],
                (((1,), (1,)), ((), ())),
                preferred_element_type=jnp.float32)
            h_ref[s - 1] = jnp.maximum(h2t, 0.0).T.astype(jnp.bfloat16)


def _tok_body(flat_ref, wt_ref, bt_ref, out_ref):
    out = lax.dot_general(
        flat_ref[...], wt_ref[...].astype(jnp.bfloat16),
        dimension_numbers=(((1,), (1,)), ((), ())),
        preferred_element_type=jnp.float32)
    out_ref[...] = out + bt_ref[...]


def kernel(x, adj, W1, b1, W2, b2, Wt, bt):
    B, N, F_IN = x.shape
    F_OUT = W1.shape[0]
    w1t = W1.T
    b1r = b1.reshape(1, F_OUT)
    b2c = b2.reshape(F_OUT, 1)
    btr = bt.reshape(1, F_OUT)

    const = lambda shape: pl.BlockSpec(shape, lambda: tuple(0 for _ in shape))
    h = pl.pallas_call(
        _gcn_body,
        in_specs=[
            const((F_IN, F_OUT)),
            const((1, F_OUT)),
            const((F_OUT, F_OUT)),
            const((F_OUT, 1)),
            pl.BlockSpec(memory_space=pl.ANY),
            pl.BlockSpec(memory_space=pl.ANY),
        ],
        out_specs=pl.BlockSpec((B, N, F_OUT), lambda: (0, 0, 0)),
        out_shape=jax.ShapeDtypeStruct((B, N, F_OUT), jnp.bfloat16),
        scratch_shapes=[
            pltpu.VMEM((4, N, N), jnp.float32),
            pltpu.VMEM((2, N, N), jnp.bfloat16),
            pltpu.VMEM((3, N, F_IN), jnp.float32),
            pltpu.VMEM((2, F_OUT, N), jnp.float32),
            pltpu.SemaphoreType.DMA((4,)),
            pltpu.SemaphoreType.DMA((3,)),
        ],
    )(w1t, b1r, W2, b2c, x, adj)

    flat = h.reshape(B, N * F_OUT)
    return pl.pallas_call(
        _tok_body,
        in_specs=[
            pl.BlockSpec((B, N * F_OUT), lambda: (0, 0)),
            pl.BlockSpec((F_OUT, N * F_OUT), lambda: (0, 0)),
            pl.BlockSpec((1, F_OUT), lambda: (0, 0)),
        ],
        out_specs=pl.BlockSpec((B, F_OUT), lambda: (0, 0)),
        out_shape=jax.ShapeDtypeStruct((B, F_OUT), jnp.float32),
    )(flat, Wt, btr)


# speedup vs baseline: 1.0080x; 1.0080x over previous
import jax
import jax.numpy as jnp
from jax import lax
from jax.experimental import pallas as pl
from jax.experimental.pallas import tpu as pltpu


def _gcn_body(w1t_ref, b1_ref, w2_ref, b2c_ref, x_hbm, adj_hbm, h_ref,
              abuf, xbuf, g2buf, asems, xsems):
    B = h_ref.shape[0]

    def start_batch(i):
        pltpu.make_async_copy(
            adj_hbm.at[i], abuf.at[i % 4], asems.at[i % 4]).start()
        pltpu.make_async_copy(
            x_hbm.at[i], xbuf.at[i % 3], xsems.at[i % 3]).start()

    def wait_batch(i):
        pltpu.make_async_copy(
            adj_hbm.at[i], abuf.at[i % 4], asems.at[i % 4]).wait()
        pltpu.make_async_copy(
            x_hbm.at[i], xbuf.at[i % 3], xsems.at[i % 3]).wait()

    start_batch(0)
    if B > 1:
        start_batch(1)
    for s in range(B + 1):
        if s < B:
            if s + 2 < B:
                start_batch(s + 2)
            wait_batch(s)
            g1 = jnp.dot(xbuf[s % 3], w1t_ref[...],
                         preferred_element_type=jnp.float32) + b1_ref[...]
            h1t = lax.dot_general(g1, abuf[s % 4], (((0,), (1,)), ((), ())),
                                  preferred_element_type=jnp.float32)
            r1t = jnp.maximum(h1t, 0.0)
            g2buf[s % 2] = jnp.dot(
                w2_ref[...], r1t,
                preferred_element_type=jnp.float32) + b2c_ref[...]
        if s >= 1:
            h2t = lax.dot_general(
                g2buf[(s - 1) % 2], abuf[(s - 1) % 4],
                (((1,), (1,)), ((), ())),
                preferred_element_type=jnp.float32)
            h_ref[s - 1] = jnp.maximum(h2t, 0.0).astype(jnp.bfloat16).T


def _tok_body(flat_ref, wt_ref, bt_ref, out_ref):
    out = lax.dot_general(
        flat_ref[...], wt_ref[...].astype(jnp.bfloat16),
        dimension_numbers=(((1,), (1,)), ((), ())),
        preferred_element_type=jnp.float32)
    out_ref[...] = out + bt_ref[...]


def kernel(x, adj, W1, b1, W2, b2, Wt, bt):
    B, N, F_IN = x.shape
    F_OUT = W1.shape[0]
    w1t = W1.T
    b1r = b1.reshape(1, F_OUT)
    b2c = b2.reshape(F_OUT, 1)
    btr = bt.reshape(1, F_OUT)

    const = lambda shape: pl.BlockSpec(shape, lambda: tuple(0 for _ in shape))
    h = pl.pallas_call(
        _gcn_body,
        in_specs=[
            const((F_IN, F_OUT)),
            const((1, F_OUT)),
            const((F_OUT, F_OUT)),
            const((F_OUT, 1)),
            pl.BlockSpec(memory_space=pl.ANY),
            pl.BlockSpec(memory_space=pl.ANY),
        ],
        out_specs=pl.BlockSpec((B, N, F_OUT), lambda: (0, 0, 0)),
        out_shape=jax.ShapeDtypeStruct((B, N, F_OUT), jnp.bfloat16),
        scratch_shapes=[
            pltpu.VMEM((4, N, N), jnp.float32),
            pltpu.VMEM((3, N, F_IN), jnp.float32),
            pltpu.VMEM((2, F_OUT, N), jnp.float32),
            pltpu.SemaphoreType.DMA((4,)),
            pltpu.SemaphoreType.DMA((3,)),
        ],
    )(w1t, b1r, W2, b2c, x, adj)

    flat = h.reshape(B, N * F_OUT)
    return pl.pallas_call(
        _tok_body,
        in_specs=[
            pl.BlockSpec((B, N * F_OUT), lambda: (0, 0)),
            pl.BlockSpec((F_OUT, N * F_OUT), lambda: (0, 0)),
            pl.BlockSpec((1, F_OUT), lambda: (0, 0)),
        ],
        out_specs=pl.BlockSpec((B, F_OUT), lambda: (0, 0)),
        out_shape=jax.ShapeDtypeStruct((B, F_OUT), jnp.float32),
    )(flat, Wt, btr)
